# baseline (speedup 1.0000x reference)
import jax
import jax.numpy as jnp
from jax.experimental import pallas as pl
from jax.experimental.pallas import tpu as pltpu

_B, _N, _I, _O = 16, 1024, 256, 256


def _fused_kernel(adj_ref, inv_deg_ref, x_ref, wa_ref, ocols_ref, weight_ref,
                  bias_ref, w_ih_ref, b_ih_ref, b_hh_ref, out_ref, m_s):
    n = _N

    @pl.when(pl.program_id(0) == 0)
    def _init():
        r = jax.lax.broadcasted_iota(jnp.int32, (n, n), 0)
        c = jax.lax.broadcasted_iota(jnp.int32, (n, n), 1)
        band = (r >= n - 2) | (c >= n - 2)
        m_s[...] = jnp.where(band, 1.0, wa_ref[...]).astype(jnp.bfloat16)

    for s in range(2):
        _one_batch(s, adj_ref, inv_deg_ref, x_ref, ocols_ref, weight_ref,
                   bias_ref, w_ih_ref, b_ih_ref, b_hh_ref, out_ref, m_s)


def _one_batch(s, adj_ref, inv_deg_ref, x_ref, ocols_ref, weight_ref,
               bias_ref, w_ih_ref, b_ih_ref, b_hh_ref, out_ref, m_s):
    n, i_dim, o_dim = _N, _I, _O
    adj = adj_ref[s]
    x = x_ref[s]
    x_bf = x.astype(jnp.bfloat16)
    xt_bf = x_bf.T

    w_adj = adj.astype(jnp.bfloat16) * m_s[...]
    sum_xt = jax.lax.dot_general(
        xt_bf, w_adj, (((1,), (0,)), ((), ())),
        preferred_element_type=jnp.float32)
    sum_x_bf = sum_xt.astype(jnp.bfloat16).T
    mean_x = jax.lax.dot_general(
        inv_deg_ref[s].astype(jnp.bfloat16), sum_x_bf,
        (((1,), (0,)), ((), ())),
        preferred_element_type=jnp.float32)

    cols = adj[:, n - 2:n]
    ocols = ocols_ref[...]
    vals = jnp.where(cols == 1.0, ocols, jnp.inf)
    minv = jnp.min(vals, axis=0, keepdims=True)
    iota = jax.lax.broadcasted_iota(jnp.int32, (n, 2), 0)
    last_idx = jnp.max(jnp.where(vals == minv, iota, -1), axis=0,
                       keepdims=True)
    has = (minv < jnp.inf).astype(jnp.float32)
    onehot = (iota == last_idx).astype(jnp.bfloat16)
    selx_t = jax.lax.dot_general(
        xt_bf, onehot, (((1,), (0,)), ((), ())),
        preferred_element_type=jnp.float32)
    gates_t = jax.lax.dot_general(
        w_ih_ref[...], selx_t, (((1,), (0,)), ((), ())),
        preferred_element_type=jnp.float32)
    gates_t = gates_t + b_ih_ref[...] + b_hh_ref[...]
    gi = gates_t[0:i_dim]
    gg = gates_t[2 * i_dim:3 * i_dim]
    go = gates_t[3 * i_dim:4 * i_dim]
    c = jax.nn.sigmoid(gi) * jnp.tanh(gg)
    h_t = jax.nn.sigmoid(go) * jnp.tanh(c)
    h_t = h_t * has

    weight_bf = weight_ref[...].astype(jnp.bfloat16)
    new_x = jnp.dot(mean_x.astype(jnp.bfloat16), weight_bf,
                    preferred_element_type=jnp.float32)
    new_h = jax.lax.dot_general(
        h_t, weight_ref[...], (((0,), (0,)), ((), ())),
        preferred_element_type=jnp.float32)
    self_x = jnp.dot(x_bf, bias_ref[...].astype(jnp.bfloat16),
                     preferred_element_type=jnp.float32)

    ones = jnp.full((o_dim, 1), 1.0, jnp.float32)
    s2 = new_x * new_x + self_x * self_x
    ss = jnp.dot(s2, ones, preferred_element_type=jnp.float32)
    ssh = (jnp.sum(new_h * new_h, axis=1, keepdims=True)
           + jnp.sum(self_x[n - 2:n] * self_x[n - 2:n], axis=1,
                     keepdims=True))
    inv = jax.lax.rsqrt(jnp.maximum(ss, 1e-24))
    invh = jax.lax.rsqrt(jnp.maximum(ssh, 1e-24))
    out_ref[s, :, 0:o_dim] = new_x * inv
    out_ref[s, :, o_dim:2 * o_dim] = self_x * inv
    out_ref[s, n - 2:n, 0:o_dim] = new_h * invh
    out_ref[s, n - 2:n, o_dim:2 * o_dim] = self_x[n - 2:n] * invh


def kernel(x, adj, inv_deg, weight, bias, weight_adj, order_adj, w_ih, w_hh,
           b_ih, b_hh):
    n = _N
    ocols = order_adj[:, n - 2:]
    b_ih2 = b_ih.reshape(4 * _I, 1)
    b_hh2 = b_hh.reshape(4 * _I, 1)

    grid = (_B // 2,)
    out = pl.pallas_call(
        _fused_kernel,
        grid=grid,
        in_specs=[
            pl.BlockSpec((2, n, n), lambda b_: (b_, 0, 0)),
            pl.BlockSpec((2, n, n), lambda b_: (b_, 0, 0)),
            pl.BlockSpec((2, n, _I), lambda b_: (b_, 0, 0)),
            pl.BlockSpec((n, n), lambda b_: (0, 0)),
            pl.BlockSpec((n, 2), lambda b_: (0, 0)),
            pl.BlockSpec((_I, _O), lambda b_: (0, 0)),
            pl.BlockSpec((_I, _O), lambda b_: (0, 0)),
            pl.BlockSpec((4 * _I, _I), lambda b_: (0, 0)),
            pl.BlockSpec((4 * _I, 1), lambda b_: (0, 0)),
            pl.BlockSpec((4 * _I, 1), lambda b_: (0, 0)),
        ],
        out_specs=pl.BlockSpec((2, n, 2 * _O), lambda b_: (b_, 0, 0)),
        out_shape=jax.ShapeDtypeStruct((_B, n, 2 * _O), jnp.float32),
        scratch_shapes=[pltpu.VMEM((n, n), jnp.bfloat16)],
    )(adj, inv_deg, x, weight_adj, ocols, weight, bias, w_ih, b_ih2, b_hh2)
    return out


# speedup vs baseline: 1.0393x; 1.0393x over previous
import jax
import jax.numpy as jnp
from jax.experimental import pallas as pl
from jax.experimental.pallas import tpu as pltpu

_B, _N, _I, _O = 16, 1024, 256, 256


def _fused_kernel(adj_ref, inv_deg_ref, x_ref, wa_ref, oadj_ref, weight_ref,
                  bias_ref, w_ih_ref, b_ih_ref, b_hh_ref, out_ref, m_s):
    n, i_dim, o_dim = _N, _I, _O

    @pl.when(pl.program_id(0) == 0)
    def _init():
        r = jax.lax.broadcasted_iota(jnp.int32, (n, n), 0)
        c = jax.lax.broadcasted_iota(jnp.int32, (n, n), 1)
        band = (r >= n - 2) | (c >= n - 2)
        m_s[...] = jnp.where(band, 1.0, wa_ref[...]).astype(jnp.bfloat16)

    adj = adj_ref[0]
    x = x_ref[0]
    x_bf = x.astype(jnp.bfloat16)

    w_adj = adj.astype(jnp.bfloat16) * m_s[...]
    sum_x = jax.lax.dot_general(
        w_adj, x_bf, (((0,), (0,)), ((), ())),
        preferred_element_type=jnp.float32)
    mean_x = jax.lax.dot_general(
        inv_deg_ref[0].astype(jnp.bfloat16), sum_x.astype(jnp.bfloat16),
        (((1,), (0,)), ((), ())),
        preferred_element_type=jnp.float32)

    cols = adj[:, n - 2:n]
    ocols = oadj_ref[:, n - 2:n]
    neigh = cols == 1.0
    vals = jnp.where(neigh, ocols, jnp.inf)
    minv = jnp.min(vals, axis=0, keepdims=True)
    iota = jax.lax.broadcasted_iota(jnp.int32, (n, 2), 0)
    cand = neigh & (vals == minv)
    last_idx = jnp.max(jnp.where(cand, iota, -1), axis=0, keepdims=True)
    has = jnp.max(jnp.where(neigh, 1.0, 0.0), axis=0, keepdims=True)
    onehot = (iota == last_idx).astype(jnp.float32)
    selx = jax.lax.dot_general(
        onehot, x, (((0,), (0,)), ((), ())),
        preferred_element_type=jnp.float32)
    gates = jax.lax.dot_general(
        selx, w_ih_ref[...], (((1,), (1,)), ((), ())),
        preferred_element_type=jnp.float32)
    gates = gates + b_ih_ref[...] + b_hh_ref[...]
    gi = gates[:, 0:i_dim]
    gg = gates[:, 2 * i_dim:3 * i_dim]
    go = gates[:, 3 * i_dim:4 * i_dim]
    c = jax.nn.sigmoid(gi) * jnp.tanh(gg)
    h = jax.nn.sigmoid(go) * jnp.tanh(c)
    h = h * has.T

    weight_bf = weight_ref[...].astype(jnp.bfloat16)
    new_x = jnp.dot(mean_x.astype(jnp.bfloat16), weight_bf,
                    preferred_element_type=jnp.float32)
    new_h = jnp.dot(h, weight_ref[...],
                    preferred_element_type=jnp.float32)
    self_x = jnp.dot(x_bf, bias_ref[...].astype(jnp.bfloat16),
                     preferred_element_type=jnp.float32)

    ss = (jnp.sum(new_x * new_x, axis=1, keepdims=True)
          + jnp.sum(self_x * self_x, axis=1, keepdims=True))
    ssh = (jnp.sum(new_h * new_h, axis=1, keepdims=True)
           + jnp.sum(self_x[n - 2:n] * self_x[n - 2:n], axis=1,
                     keepdims=True))
    inv = jax.lax.rsqrt(jnp.maximum(ss, 1e-24))
    invh = jax.lax.rsqrt(jnp.maximum(ssh, 1e-24))
    out_ref[0, :, 0:o_dim] = new_x * inv
    out_ref[0, :, o_dim:2 * o_dim] = self_x * inv
    out_ref[0, n - 2:n, 0:o_dim] = new_h * invh
    out_ref[0, n - 2:n, o_dim:2 * o_dim] = self_x[n - 2:n] * invh


def kernel(x, adj, inv_deg, weight, bias, weight_adj, order_adj, w_ih, w_hh,
           b_ih, b_hh):
    n = _N
    b_ih2 = b_ih.reshape(1, 4 * _I)
    b_hh2 = b_hh.reshape(1, 4 * _I)

    grid = (_B,)
    out = pl.pallas_call(
        _fused_kernel,
        grid=grid,
        in_specs=[
            pl.BlockSpec((1, n, n), lambda b_: (b_, 0, 0)),
            pl.BlockSpec((1, n, n), lambda b_: (b_, 0, 0)),
            pl.BlockSpec((1, n, _I), lambda b_: (b_, 0, 0)),
            pl.BlockSpec((n, n), lambda b_: (0, 0)),
            pl.BlockSpec((n, n), lambda b_: (0, 0)),
            pl.BlockSpec((_I, _O), lambda b_: (0, 0)),
            pl.BlockSpec((_I, _O), lambda b_: (0, 0)),
            pl.BlockSpec((4 * _I, _I), lambda b_: (0, 0)),
            pl.BlockSpec((1, 4 * _I), lambda b_: (0, 0)),
            pl.BlockSpec((1, 4 * _I), lambda b_: (0, 0)),
        ],
        out_specs=pl.BlockSpec((1, n, 2 * _O), lambda b_: (b_, 0, 0)),
        out_shape=jax.ShapeDtypeStruct((_B, n, 2 * _O), jnp.float32),
        scratch_shapes=[pltpu.VMEM((n, n), jnp.bfloat16)],
    )(adj, inv_deg, x, weight_adj, order_adj, weight, bias, w_ih, b_ih2, b_hh2)
    return out
